# baseline (speedup 1.0000x reference)
import jax
import jax.numpy as jnp
from jax import lax
from jax.experimental import pallas as pl
from jax.experimental.pallas import tpu as pltpu
from jax.experimental.pallas import tpu_sc as plsc

N_ROWS = 320000
N_FEAT = 128
HALF = N_FEAT // 2
N_SEG = 10000
NC = 2
NS = 16
SUB = 128
G = 2
GROWS = G * SUB
NSLOT = 4
BASE_N = 92
SC_SUB = BASE_N * NS
SC_ROWS = SC_SUB * SUB
GROUPS = BASE_N // G
SEG_PER_TILE = N_SEG // NS

TCR = 512
TCW = 128
TC_OFF_BLKS = SC_ROWS // TCR
TC_NB = (N_ROWS - SC_ROWS) // TCR
NSEGP = N_SEG + TCW
BIG = 2**30


def _sc_body(x_hbm, idx2_hbm, zeros_hbm, out_hbm, acc, xbuf, iall, lsem,
             ssem):
    c = lax.axis_index("c")
    s = lax.axis_index("s")
    base_sub = s * BASE_N

    def start_load(g, slot):
        sub = base_sub + g * G
        pltpu.async_copy(
            x_hbm.at[pl.ds(sub * SUB, GROWS), pl.ds(c * HALF, HALF)],
            xbuf.at[slot], lsem.at[slot])

    def wait_load(slot):
        pltpu.make_async_copy(
            x_hbm.at[pl.ds(0, GROWS), pl.ds(0, HALF)],
            xbuf.at[slot], lsem.at[slot]).wait()

    def fire_scatters(g, slot):
        for k in range(G):
            pltpu.async_copy(
                xbuf.at[slot, pl.ds(k * SUB, SUB)],
                acc.at[iall.at[g * G + k]], ssem.at[slot], add=True)

    def drain_scatters(slot):
        for k in range(G):
            pltpu.make_async_copy(
                xbuf.at[slot, pl.ds(k * SUB, SUB)],
                acc.at[iall.at[0]], ssem.at[slot]).wait()

    for p in range(2):
        start_load(p, p)

    pltpu.sync_copy(idx2_hbm.at[pl.ds(base_sub, BASE_N)], iall)

    pltpu.sync_copy(zeros_hbm, acc.at[pl.ds(s * SEG_PER_TILE, SEG_PER_TILE)])
    plsc.subcore_barrier()

    def step(g, b):
        wait_load(b)
        fire_scatters(g, b)
        s2 = (b + 2) % NSLOT

        @pl.when(g >= 2)
        def _():
            drain_scatters(s2)

        @pl.when(g + 2 < GROUPS)
        def _():
            start_load(g + 2, s2)

    def outer(gbase, carry):
        for b in range(NSLOT):
            step(gbase + b, b)
        return carry

    lax.fori_loop(0, GROUPS // NSLOT, lambda i, cr: outer(i * NSLOT, cr), 0)

    for r in range(GROUPS - GROUPS // NSLOT * NSLOT):
        g = GROUPS // NSLOT * NSLOT + r
        step(g, g % NSLOT)

    drain_scatters((GROUPS - 2) % NSLOT)
    drain_scatters((GROUPS - 1) % NSLOT)

    plsc.subcore_barrier()
    pltpu.sync_copy(
        acc.at[pl.ds(s * SEG_PER_TILE, SEG_PER_TILE)],
        out_hbm.at[pl.ds(s * SEG_PER_TILE, SEG_PER_TILE),
                   pl.ds(c * HALF, HALF)],
    )


def _tc_body(idx_ref, x_ref, o_ref):
    i = pl.program_id(0)

    @pl.when(i == 0)
    def _():
        o_ref[...] = jnp.zeros_like(o_ref)

    first = idx_ref[0, 0, 0]
    offs = idx_ref[0] - first

    def body(base):
        sel = (offs >= base) & (offs < base + TCW)
        oh = (offs - base) == lax.broadcasted_iota(
            jnp.int32, (TCR, TCW), 1)
        ohf = jnp.where(sel, oh.astype(jnp.float32), 0.0)
        contrib = lax.dot_general(
            ohf, x_ref[...], (((0,), (0,)), ((), ())),
            preferred_element_type=jnp.float32,
            precision=lax.Precision.HIGHEST)
        row = first + base
        o_ref[pl.ds(row, TCW), :] = o_ref[pl.ds(row, TCW), :] + contrib
        return jnp.min(jnp.where(offs >= base + TCW, offs, BIG))

    lax.while_loop(lambda b: b < BIG, body, jnp.int32(0))


def _add_body(a_ref, b_ref, o_ref):
    o_ref[...] = a_ref[...] + b_ref[...]


def kernel(x, idx):
    idx32 = idx.astype(jnp.int32)
    idx2d = idx32.reshape(N_ROWS // SUB, SUB)
    idx3d = idx32.reshape(N_ROWS // TCR, TCR, 1)
    zeros = jnp.zeros((SEG_PER_TILE, HALF), jnp.float32)

    out_sc = pl.kernel(
        _sc_body,
        out_type=jax.ShapeDtypeStruct((N_SEG, N_FEAT), jnp.float32),
        mesh=plsc.VectorSubcoreMesh(core_axis_name="c", subcore_axis_name="s"),
        compiler_params=pltpu.CompilerParams(use_tc_tiling_on_sc=False),
        scratch_types=[
            pltpu.VMEM_SHARED((N_SEG, HALF), jnp.float32),
            pltpu.VMEM((NSLOT, GROWS, HALF), jnp.float32),
            pltpu.VMEM((BASE_N, SUB), jnp.int32),
            pltpu.SemaphoreType.DMA((NSLOT,)),
            pltpu.SemaphoreType.DMA((NSLOT,)),
        ],
    )(x, idx2d, zeros)

    part_tc = pl.pallas_call(
        _tc_body,
        grid=(TC_NB,),
        in_specs=[
            pl.BlockSpec((1, TCR, 1), lambda i: (TC_OFF_BLKS + i, 0, 0)),
            pl.BlockSpec((TCR, N_FEAT), lambda i: (TC_OFF_BLKS + i, 0)),
        ],
        out_specs=pl.BlockSpec((NSEGP, N_FEAT), lambda i: (0, 0)),
        out_shape=jax.ShapeDtypeStruct((NSEGP, N_FEAT), jnp.float32),
    )(idx3d, x)

    blk = 1000
    out = pl.pallas_call(
        _add_body,
        grid=(N_SEG // blk,),
        in_specs=[
            pl.BlockSpec((blk, N_FEAT), lambda i: (i, 0)),
            pl.BlockSpec((blk, N_FEAT), lambda i: (i, 0)),
        ],
        out_specs=pl.BlockSpec((blk, N_FEAT), lambda i: (i, 0)),
        out_shape=jax.ShapeDtypeStruct((N_SEG, N_FEAT), jnp.float32),
    )(out_sc, part_tc)
    return out


# speedup vs baseline: 3.8206x; 3.8206x over previous
import jax
import jax.numpy as jnp
from jax import lax
from jax.experimental import pallas as pl
from jax.experimental.pallas import tpu as pltpu
from jax.experimental.pallas import tpu_sc as plsc

N_ROWS = 320000
N_FEAT = 128
HALF = N_FEAT // 2
N_SEG = 10000
NC = 2
NS = 16
SUB = 128
G = 2
GROWS = G * SUB
NSLOT = 4
TOTAL_SUB = N_ROWS // SUB
BASE_N = TOTAL_SUB // NS
EXTRA = TOTAL_SUB % NS
GROUPS = BASE_N // G
SEG_PER_TILE = N_SEG // NS


def _sc_body(x_hbm, idx2_hbm, zeros_hbm, out_hbm, acc, xbuf, iall, lsem,
             ssem):
    c = lax.axis_index("c")
    s = lax.axis_index("s")
    base_sub = s * BASE_N + jnp.minimum(s, EXTRA)

    def start_load(g, slot):
        sub = base_sub + g * G
        pltpu.async_copy(
            x_hbm.at[pl.ds(sub * SUB, GROWS), pl.ds(c * HALF, HALF)],
            xbuf.at[slot], lsem.at[slot])

    def wait_load(slot):
        pltpu.make_async_copy(
            x_hbm.at[pl.ds(0, GROWS), pl.ds(0, HALF)],
            xbuf.at[slot], lsem.at[slot]).wait()

    def fire_scatters(g, slot):
        for k in range(G):
            pltpu.async_copy(
                xbuf.at[slot, pl.ds(k * SUB, SUB)],
                acc.at[iall.at[g * G + k]], ssem.at[slot], add=True)

    def drain_scatters(slot):
        for k in range(G):
            pltpu.make_async_copy(
                xbuf.at[slot, pl.ds(k * SUB, SUB)],
                acc.at[iall.at[0]], ssem.at[slot]).wait()

    for p in range(2):
        start_load(p, p)

    pltpu.sync_copy(idx2_hbm.at[pl.ds(base_sub, BASE_N)],
                    iall.at[pl.ds(0, BASE_N)])

    @pl.when(s < EXTRA)
    def _():
        pltpu.sync_copy(idx2_hbm.at[pl.ds(base_sub + BASE_N, 1)],
                        iall.at[pl.ds(BASE_N, 1)])

    pltpu.sync_copy(zeros_hbm, acc.at[pl.ds(s * SEG_PER_TILE, SEG_PER_TILE)])
    plsc.subcore_barrier()

    def step(g, b):
        wait_load(b)
        fire_scatters(g, b)
        s2 = (b + 2) % NSLOT

        @pl.when(g >= 2)
        def _():
            drain_scatters(s2)

        @pl.when(g + 2 < GROUPS)
        def _():
            start_load(g + 2, s2)

    def outer(gbase, carry):
        for b in range(NSLOT):
            step(gbase + b, b)
        return carry

    lax.fori_loop(0, GROUPS // NSLOT, lambda i, cr: outer(i * NSLOT, cr), 0)

    for r in range(GROUPS - GROUPS // NSLOT * NSLOT):
        g = GROUPS // NSLOT * NSLOT + r
        step(g, g % NSLOT)

    drain_scatters((GROUPS - 2) % NSLOT)
    drain_scatters((GROUPS - 1) % NSLOT)

    @pl.when(s < EXTRA)
    def _():
        sub = base_sub + BASE_N
        pltpu.sync_copy(
            x_hbm.at[pl.ds(sub * SUB, SUB), pl.ds(c * HALF, HALF)],
            xbuf.at[0, pl.ds(0, SUB)])
        pltpu.sync_copy(xbuf.at[0, pl.ds(0, SUB)],
                        acc.at[iall.at[BASE_N]], add=True)

    plsc.subcore_barrier()
    pltpu.sync_copy(
        acc.at[pl.ds(s * SEG_PER_TILE, SEG_PER_TILE)],
        out_hbm.at[pl.ds(s * SEG_PER_TILE, SEG_PER_TILE),
                   pl.ds(c * HALF, HALF)],
    )


def kernel(x, idx):
    idx2d = idx.astype(jnp.int32).reshape(TOTAL_SUB, SUB)
    zeros = jnp.zeros((SEG_PER_TILE, HALF), jnp.float32)

    out = pl.kernel(
        _sc_body,
        out_type=jax.ShapeDtypeStruct((N_SEG, N_FEAT), jnp.float32),
        mesh=plsc.VectorSubcoreMesh(core_axis_name="c", subcore_axis_name="s"),
        compiler_params=pltpu.CompilerParams(use_tc_tiling_on_sc=False),
        scratch_types=[
            pltpu.VMEM_SHARED((N_SEG, HALF), jnp.float32),
            pltpu.VMEM((NSLOT, GROWS, HALF), jnp.float32),
            pltpu.VMEM((BASE_N + 1, SUB), jnp.int32),
            pltpu.SemaphoreType.DMA((NSLOT,)),
            pltpu.SemaphoreType.DMA((NSLOT,)),
        ],
    )(x, idx2d, zeros)
    return out
